# baseline (speedup 1.0000x reference)
import functools

import jax
import jax.numpy as jnp
from jax import lax
from jax.experimental import pallas as pl
from jax.experimental.pallas import tpu as pltpu
from jax.experimental.pallas import tpu_sc as plsc

_B, _T, _Q = 32, 1024, 8
_K = 8192
_D = 128
_MOTION_DIM = 66
_J = 22

_N = _B * _T
_NC, _NS, _L = 2, 16, 16
_NW = _NC * _NS
_TOK_W = _N // _NW
_CH_T = 8
_ROWS = _CH_T * _Q
_NCHUNK = _TOK_W // _CH_T
_NBUF = 8
_WBUF = 2

_sc_mesh = plsc.VectorSubcoreMesh(core_axis_name="c", subcore_axis_name="s")


@functools.partial(
    pl.kernel,
    out_type=jax.ShapeDtypeStruct((_N, _D), jnp.float32),
    mesh=_sc_mesh,
    scratch_types=[
        pltpu.VMEM((_TOK_W * _Q,), jnp.int32),
        pltpu.VMEM((_NBUF, _ROWS, _D), jnp.float32),
        pltpu.VMEM((_WBUF, _CH_T, _D), jnp.float32),
    ] + [pltpu.SemaphoreType.DMA] * (_NBUF + _WBUF),
)
def _gather_sum(idx_hbm, codebook_hbm, feat_hbm, idx_v, rows_v, feat_v,
                *sems):
    gsem = sems[:_NBUF]
    wsem = sems[_NBUF:]
    wid = lax.axis_index("s") * _NC + lax.axis_index("c")
    tok0 = wid * _TOK_W

    pltpu.sync_copy(idx_hbm.at[pl.ds(wid * _TOK_W * _Q, _TOK_W * _Q)], idx_v)

    for b in range(_NBUF):
        pltpu.async_copy(codebook_hbm.at[idx_v.at[pl.ds(b * _ROWS, _ROWS)]],
                         rows_v.at[b], gsem[b])

    def _chunk(j, b, fb):
        pltpu.make_async_copy(
            codebook_hbm.at[pl.ds(0, _ROWS)], rows_v.at[b], gsem[b]).wait()

        @pl.when(j >= _WBUF)
        def _():
            pltpu.make_async_copy(
                feat_v.at[fb], feat_hbm.at[pl.ds(0, _CH_T)], wsem[fb]).wait()

        def _tok(t2, carry):
            for u in range(2):
                t = t2 * 2 + u
                r0 = t * _Q
                acc = [rows_v[b, r0, pl.ds(s * _L, _L)]
                       for s in range(_D // _L)]
                for q in range(1, _Q):
                    for s in range(_D // _L):
                        acc[s] = acc[s] + rows_v[b, r0 + q, pl.ds(s * _L, _L)]
                for s in range(_D // _L):
                    feat_v[fb, t, pl.ds(s * _L, _L)] = acc[s]
            return carry

        lax.fori_loop(0, _CH_T // 2, _tok, 0)

        @pl.when(j + _NBUF < _NCHUNK)
        def _():
            pltpu.async_copy(
                codebook_hbm.at[idx_v.at[pl.ds((j + _NBUF) * _ROWS, _ROWS)]],
                rows_v.at[b], gsem[b])

        pltpu.async_copy(
            feat_v.at[fb], feat_hbm.at[pl.ds(tok0 + j * _CH_T, _CH_T)],
            wsem[fb])

    def _outer(i, carry):
        for b in range(_NBUF):
            _chunk(i * _NBUF + b, b, b % _WBUF)
        return carry

    lax.fori_loop(0, _NCHUNK // _NBUF, _outer, 0)

    for fb in range(_WBUF):
        pltpu.make_async_copy(
            feat_v.at[fb], feat_hbm.at[pl.ds(0, _CH_T)], wsem[fb]).wait()


def _mlp_body(feat_ref, w1_ref, b1_ref, w2_ref, b2_ref, out_ref):
    h = lax.dot_general(feat_ref[...], w1_ref[...], (((1,), (0,)), ((), ())),
                        preferred_element_type=jnp.float32)
    h = jnp.maximum(h + b1_ref[...], 0.0)
    out_ref[...] = lax.dot_general(w2_ref[...], h, (((0,), (1,)), ((), ())),
                                   preferred_element_type=jnp.float32) + b2_ref[...]


_BT = 2048

_mlp = pl.pallas_call(
    _mlp_body,
    grid=(_N // _BT,),
    in_specs=[
        pl.BlockSpec((_BT, _D), lambda i: (i, 0)),
        pl.BlockSpec((_D, _D), lambda i: (0, 0)),
        pl.BlockSpec((1, _D), lambda i: (0, 0)),
        pl.BlockSpec((_D, _MOTION_DIM), lambda i: (0, 0)),
        pl.BlockSpec((_MOTION_DIM, 1), lambda i: (0, 0)),
    ],
    out_specs=pl.BlockSpec((_MOTION_DIM, _BT), lambda i: (0, i)),
    out_shape=jax.ShapeDtypeStruct((_MOTION_DIM, _N), jnp.float32),
)


def kernel(tokens, codebook, W1, b1, W2, b2):
    flat = tokens.astype(jnp.int32).reshape(_N * _Q)
    offs = lax.rem(lax.iota(jnp.int32, _N * _Q), _Q) * _K
    idx1d = flat + offs
    feat = _gather_sum(idx1d, codebook)
    motion_t = _mlp(feat, W1, b1.reshape(1, _D), W2,
                    b2.reshape(_MOTION_DIM, 1))
    return motion_t.T.reshape(_B, _T, _J, 3)


# speedup vs baseline: 1.0455x; 1.0455x over previous
import functools

import jax
import jax.numpy as jnp
from jax import lax
from jax.experimental import pallas as pl
from jax.experimental.pallas import tpu as pltpu
from jax.experimental.pallas import tpu_sc as plsc

_B, _T, _Q = 32, 1024, 8
_K = 8192
_D = 128
_MOTION_DIM = 66
_J = 22

_N = _B * _T
_NC, _NS, _L = 2, 16, 16
_NW = _NC * _NS
_TOK_W = _N // _NW
_CH_T = 16
_ROWS = _CH_T * _Q
_NCHUNK = _TOK_W // _CH_T
_NBUF = 4
_WBUF = 2

_sc_mesh = plsc.VectorSubcoreMesh(core_axis_name="c", subcore_axis_name="s")


@functools.partial(
    pl.kernel,
    out_type=jax.ShapeDtypeStruct((_N, _D), jnp.float32),
    mesh=_sc_mesh,
    scratch_types=[
        pltpu.VMEM((_TOK_W * _Q,), jnp.int32),
        pltpu.VMEM((_NBUF, _ROWS, _D), jnp.float32),
        pltpu.VMEM((_WBUF, _CH_T, _D), jnp.float32),
    ] + [pltpu.SemaphoreType.DMA] * (_NBUF + _WBUF),
)
def _gather_sum(idx_hbm, codebook_hbm, feat_hbm, idx_v, rows_v, feat_v,
                *sems):
    gsem = sems[:_NBUF]
    wsem = sems[_NBUF:]
    wid = lax.axis_index("s") * _NC + lax.axis_index("c")
    tok0 = wid * _TOK_W

    pltpu.sync_copy(idx_hbm.at[pl.ds(wid * _TOK_W * _Q, _TOK_W * _Q)], idx_v)

    for b in range(_NBUF):
        pltpu.async_copy(codebook_hbm.at[idx_v.at[pl.ds(b * _ROWS, _ROWS)]],
                         rows_v.at[b], gsem[b])

    def _chunk(j, b, fb):
        pltpu.make_async_copy(
            codebook_hbm.at[pl.ds(0, _ROWS)], rows_v.at[b], gsem[b]).wait()

        @pl.when(j >= _WBUF)
        def _():
            pltpu.make_async_copy(
                feat_v.at[fb], feat_hbm.at[pl.ds(0, _CH_T)], wsem[fb]).wait()

        def _tok(t2, carry):
            for u in range(2):
                t = t2 * 2 + u
                r0 = t * _Q
                acc = [rows_v[b, r0, pl.ds(s * _L, _L)]
                       for s in range(_D // _L)]
                for q in range(1, _Q):
                    for s in range(_D // _L):
                        acc[s] = acc[s] + rows_v[b, r0 + q, pl.ds(s * _L, _L)]
                for s in range(_D // _L):
                    feat_v[fb, t, pl.ds(s * _L, _L)] = acc[s]
            return carry

        lax.fori_loop(0, _CH_T // 2, _tok, 0)

        @pl.when(j + _NBUF < _NCHUNK)
        def _():
            pltpu.async_copy(
                codebook_hbm.at[idx_v.at[pl.ds((j + _NBUF) * _ROWS, _ROWS)]],
                rows_v.at[b], gsem[b])

        pltpu.async_copy(
            feat_v.at[fb], feat_hbm.at[pl.ds(tok0 + j * _CH_T, _CH_T)],
            wsem[fb])

    def _outer(i, carry):
        for b in range(_NBUF):
            _chunk(i * _NBUF + b, b, b % _WBUF)
        return carry

    lax.fori_loop(0, _NCHUNK // _NBUF, _outer, 0)

    for fb in range(_WBUF):
        pltpu.make_async_copy(
            feat_v.at[fb], feat_hbm.at[pl.ds(0, _CH_T)], wsem[fb]).wait()


def _mlp_body(feat_ref, w1_ref, b1_ref, w2_ref, b2_ref, out_ref):
    h = lax.dot_general(feat_ref[...], w1_ref[...], (((1,), (0,)), ((), ())),
                        preferred_element_type=jnp.float32)
    h = jnp.maximum(h + b1_ref[...], 0.0)
    out_ref[...] = lax.dot_general(w2_ref[...], h, (((0,), (1,)), ((), ())),
                                   preferred_element_type=jnp.float32) + b2_ref[...]


_BT = 4096

_mlp = pl.pallas_call(
    _mlp_body,
    grid=(_N // _BT,),
    in_specs=[
        pl.BlockSpec((_BT, _D), lambda i: (i, 0)),
        pl.BlockSpec((_D, _D), lambda i: (0, 0)),
        pl.BlockSpec((1, _D), lambda i: (0, 0)),
        pl.BlockSpec((_D, _MOTION_DIM), lambda i: (0, 0)),
        pl.BlockSpec((_MOTION_DIM, 1), lambda i: (0, 0)),
    ],
    out_specs=pl.BlockSpec((_MOTION_DIM, _BT), lambda i: (0, i)),
    out_shape=jax.ShapeDtypeStruct((_MOTION_DIM, _N), jnp.float32),
)


def kernel(tokens, codebook, W1, b1, W2, b2):
    flat = tokens.astype(jnp.int32).reshape(_N * _Q)
    offs = lax.rem(lax.iota(jnp.int32, _N * _Q), _Q) * _K
    idx1d = flat + offs
    feat = _gather_sum(idx1d, codebook)
    motion_t = _mlp(feat, W1, b1.reshape(1, _D), W2,
                    b2.reshape(_MOTION_DIM, 1))
    return motion_t.T.reshape(_B, _T, _J, 3)


# speedup vs baseline: 1.0638x; 1.0175x over previous
import functools

import jax
import jax.numpy as jnp
from jax import lax
from jax.experimental import pallas as pl
from jax.experimental.pallas import tpu as pltpu
from jax.experimental.pallas import tpu_sc as plsc

_B, _T, _Q = 32, 1024, 8
_K = 8192
_D = 128
_MOTION_DIM = 66
_J = 22

_N = _B * _T
_NC, _NS, _L = 2, 16, 16
_NW = _NC * _NS
_TOK_W = _N // _NW
_CH_T = 16
_ROWS = _CH_T * _Q
_NCHUNK = _TOK_W // _CH_T
_NBUF = 4
_WBUF = 2

_sc_mesh = plsc.VectorSubcoreMesh(core_axis_name="c", subcore_axis_name="s")


@functools.partial(
    pl.kernel,
    out_type=jax.ShapeDtypeStruct((_N, _D), jnp.float32),
    mesh=_sc_mesh,
    scratch_types=[
        pltpu.VMEM((_TOK_W * _Q,), jnp.int32),
        pltpu.VMEM((_NBUF, _ROWS, _D), jnp.float32),
        pltpu.VMEM((_WBUF, _CH_T, _D), jnp.float32),
    ] + [pltpu.SemaphoreType.DMA] * (_NBUF + _WBUF),
)
def _gather_sum(idx_hbm, codebook_hbm, feat_hbm, idx_v, rows_v, feat_v,
                *sems):
    gsem = sems[:_NBUF]
    wsem = sems[_NBUF:]
    wid = lax.axis_index("s") * _NC + lax.axis_index("c")
    tok0 = wid * _TOK_W

    pltpu.sync_copy(idx_hbm.at[pl.ds(wid * _TOK_W * _Q, _TOK_W * _Q)], idx_v)

    for b in range(_NBUF):
        pltpu.async_copy(codebook_hbm.at[idx_v.at[pl.ds(b * _ROWS, _ROWS)]],
                         rows_v.at[b], gsem[b])

    def _chunk(j, b, fb):
        pltpu.make_async_copy(
            codebook_hbm.at[pl.ds(0, _ROWS)], rows_v.at[b], gsem[b]).wait()

        @pl.when(j >= _WBUF)
        def _():
            pltpu.make_async_copy(
                feat_v.at[fb], feat_hbm.at[pl.ds(0, _CH_T)], wsem[fb]).wait()

        def _tok(t2, carry):
            for u in range(2):
                t = t2 * 2 + u
                r0 = t * _Q
                acc = [rows_v[b, r0, pl.ds(s * _L, _L)]
                       for s in range(_D // _L)]
                for q in range(1, _Q):
                    for s in range(_D // _L):
                        acc[s] = acc[s] + rows_v[b, r0 + q, pl.ds(s * _L, _L)]
                for s in range(_D // _L):
                    feat_v[fb, t, pl.ds(s * _L, _L)] = acc[s]
            return carry

        lax.fori_loop(0, _CH_T // 2, _tok, 0)

        @pl.when(j + _NBUF < _NCHUNK)
        def _():
            pltpu.async_copy(
                codebook_hbm.at[idx_v.at[pl.ds((j + _NBUF) * _ROWS, _ROWS)]],
                rows_v.at[b], gsem[b])

        pltpu.async_copy(
            feat_v.at[fb], feat_hbm.at[pl.ds(tok0 + j * _CH_T, _CH_T)],
            wsem[fb])

    def _outer(i, carry):
        for b in range(_NBUF):
            _chunk(i * _NBUF + b, b, b % _WBUF)
        return carry

    lax.fori_loop(0, _NCHUNK // _NBUF, _outer, 0)

    for fb in range(_WBUF):
        pltpu.make_async_copy(
            feat_v.at[fb], feat_hbm.at[pl.ds(0, _CH_T)], wsem[fb]).wait()


def _mlp_body(feat_ref, w1_ref, b1_ref, w2_ref, b2_ref, out_ref):
    h = lax.dot_general(feat_ref[...], w1_ref[...], (((1,), (0,)), ((), ())),
                        preferred_element_type=jnp.float32)
    h = jnp.maximum(h + b1_ref[...], 0.0)
    out_ref[...] = lax.dot_general(w2_ref[...], h, (((0,), (1,)), ((), ())),
                                   preferred_element_type=jnp.float32) + b2_ref[...]


_BT = 8192

_mlp = pl.pallas_call(
    _mlp_body,
    grid=(_N // _BT,),
    in_specs=[
        pl.BlockSpec((_BT, _D), lambda i: (i, 0)),
        pl.BlockSpec((_D, _D), lambda i: (0, 0)),
        pl.BlockSpec((1, _D), lambda i: (0, 0)),
        pl.BlockSpec((_D, _MOTION_DIM), lambda i: (0, 0)),
        pl.BlockSpec((_MOTION_DIM, 1), lambda i: (0, 0)),
    ],
    out_specs=pl.BlockSpec((_MOTION_DIM, _BT), lambda i: (0, i)),
    out_shape=jax.ShapeDtypeStruct((_MOTION_DIM, _N), jnp.float32),
)


def kernel(tokens, codebook, W1, b1, W2, b2):
    flat = tokens.astype(jnp.int32).reshape(_N * _Q)
    offs = lax.rem(lax.iota(jnp.int32, _N * _Q), _Q) * _K
    idx1d = flat + offs
    feat = _gather_sum(idx1d, codebook)
    motion_t = _mlp(feat, W1, b1.reshape(1, _D), W2,
                    b2.reshape(_MOTION_DIM, 1))
    return motion_t.T.reshape(_B, _T, _J, 3)


# speedup vs baseline: 1.0690x; 1.0049x over previous
import functools

import jax
import jax.numpy as jnp
from jax import lax
from jax.experimental import pallas as pl
from jax.experimental.pallas import tpu as pltpu
from jax.experimental.pallas import tpu_sc as plsc

_B, _T, _Q = 32, 1024, 8
_K = 8192
_D = 128
_MOTION_DIM = 66
_J = 22

_N = _B * _T
_NC, _NS, _L = 2, 16, 16
_NW = _NC * _NS
_TOK_W = _N // _NW
_CH_T = 16
_ROWS = _CH_T * _Q
_NCHUNK = _TOK_W // _CH_T
_NBUF = 4
_WBUF = 2

_sc_mesh = plsc.VectorSubcoreMesh(core_axis_name="c", subcore_axis_name="s")


@functools.partial(
    pl.kernel,
    out_type=jax.ShapeDtypeStruct((_N, _D), jnp.float32),
    mesh=_sc_mesh,
    scratch_types=[
        pltpu.VMEM((_TOK_W * _Q,), jnp.int32),
        pltpu.VMEM((_NBUF, _ROWS, _D), jnp.float32),
        pltpu.VMEM((_WBUF, _CH_T, _D), jnp.float32),
    ] + [pltpu.SemaphoreType.DMA] * (_NBUF + _WBUF),
)
def _gather_sum(idx_hbm, codebook_hbm, feat_hbm, idx_v, rows_v, feat_v,
                *sems):
    gsem = sems[:_NBUF]
    wsem = sems[_NBUF:]
    wid = lax.axis_index("s") * _NC + lax.axis_index("c")
    tok0 = wid * _TOK_W

    pltpu.sync_copy(idx_hbm.at[pl.ds(wid * _TOK_W * _Q, _TOK_W * _Q)], idx_v)

    for b in range(_NBUF):
        pltpu.async_copy(codebook_hbm.at[idx_v.at[pl.ds(b * _ROWS, _ROWS)]],
                         rows_v.at[b], gsem[b])

    def _chunk(j, b, fb):
        pltpu.make_async_copy(
            codebook_hbm.at[pl.ds(0, _ROWS)], rows_v.at[b], gsem[b]).wait()

        @pl.when(j >= _WBUF)
        def _():
            pltpu.make_async_copy(
                feat_v.at[fb], feat_hbm.at[pl.ds(0, _CH_T)], wsem[fb]).wait()

        def _tok(t2, carry):
            for u in range(2):
                t = t2 * 2 + u
                r0 = t * _Q
                acc = [rows_v[b, r0, pl.ds(s * _L, _L)]
                       for s in range(_D // _L)]
                for q in range(1, _Q):
                    for s in range(_D // _L):
                        acc[s] = acc[s] + rows_v[b, r0 + q, pl.ds(s * _L, _L)]
                for s in range(_D // _L):
                    feat_v[fb, t, pl.ds(s * _L, _L)] = acc[s]
            return carry

        lax.fori_loop(0, _CH_T // 2, _tok, 0)

        @pl.when(j + _NBUF < _NCHUNK)
        def _():
            pltpu.async_copy(
                codebook_hbm.at[idx_v.at[pl.ds((j + _NBUF) * _ROWS, _ROWS)]],
                rows_v.at[b], gsem[b])

        pltpu.async_copy(
            feat_v.at[fb], feat_hbm.at[pl.ds(tok0 + j * _CH_T, _CH_T)],
            wsem[fb])

    def _outer(i, carry):
        for b in range(_NBUF):
            _chunk(i * _NBUF + b, b, b % _WBUF)
        return carry

    lax.fori_loop(0, _NCHUNK // _NBUF, _outer, 0)

    for fb in range(_WBUF):
        pltpu.make_async_copy(
            feat_v.at[fb], feat_hbm.at[pl.ds(0, _CH_T)], wsem[fb]).wait()


def _mlp_body(feat_ref, w1_ref, b1_ref, w2_ref, b2_ref, out_ref):
    h = lax.dot_general(feat_ref[...], w1_ref[...], (((1,), (0,)), ((), ())),
                        preferred_element_type=jnp.float32)
    h = jnp.maximum(h + b1_ref[...], 0.0)
    out_ref[...] = lax.dot_general(w2_ref[...], h, (((0,), (1,)), ((), ())),
                                   preferred_element_type=jnp.float32) + b2_ref[...]


_BT = 16384

_mlp = pl.pallas_call(
    _mlp_body,
    grid=(_N // _BT,),
    in_specs=[
        pl.BlockSpec((_BT, _D), lambda i: (i, 0)),
        pl.BlockSpec((_D, _D), lambda i: (0, 0)),
        pl.BlockSpec((1, _D), lambda i: (0, 0)),
        pl.BlockSpec((_D, _MOTION_DIM), lambda i: (0, 0)),
        pl.BlockSpec((_MOTION_DIM, 1), lambda i: (0, 0)),
    ],
    out_specs=pl.BlockSpec((_MOTION_DIM, _BT), lambda i: (0, i)),
    out_shape=jax.ShapeDtypeStruct((_MOTION_DIM, _N), jnp.float32),
)


def kernel(tokens, codebook, W1, b1, W2, b2):
    flat = tokens.astype(jnp.int32).reshape(_N * _Q)
    offs = lax.rem(lax.iota(jnp.int32, _N * _Q), _Q) * _K
    idx1d = flat + offs
    feat = _gather_sum(idx1d, codebook)
    motion_t = _mlp(feat, W1, b1.reshape(1, _D), W2,
                    b2.reshape(_MOTION_DIM, 1))
    return motion_t.T.reshape(_B, _T, _J, 3)


# speedup vs baseline: 1.2503x; 1.1696x over previous
import functools

import jax
import jax.numpy as jnp
from jax import lax
from jax.experimental import pallas as pl
from jax.experimental.pallas import tpu as pltpu
from jax.experimental.pallas import tpu_sc as plsc

_B, _T, _Q = 32, 1024, 8
_K = 8192
_D = 128
_MOTION_DIM = 66
_J = 22

_N = _B * _T
_NC, _NS, _L = 2, 16, 16
_NW = _NC * _NS
_TOK_W = _N // _NW
_CH_T = 16
_ROWS = _CH_T * _Q
_NCHUNK = _TOK_W // _CH_T
_NBUF = 4
_WBUF = 2

_sc_mesh = plsc.VectorSubcoreMesh(core_axis_name="c", subcore_axis_name="s")


@functools.partial(
    pl.kernel,
    out_type=jax.ShapeDtypeStruct((_N, _D), jnp.float32),
    mesh=_sc_mesh,
    scratch_types=[
        pltpu.VMEM((_TOK_W * _Q,), jnp.int32),
        pltpu.VMEM((_NBUF, _ROWS, _D), jnp.float32),
        pltpu.VMEM((_WBUF, _CH_T, _D), jnp.float32),
    ] + [pltpu.SemaphoreType.DMA] * (_NBUF + _WBUF),
)
def _gather_sum(idx_hbm, codebook_hbm, feat_hbm, idx_v, rows_v, feat_v,
                *sems):
    gsem = sems[:_NBUF]
    wsem = sems[_NBUF:]
    wid = lax.axis_index("s") * _NC + lax.axis_index("c")
    tok0 = wid * _TOK_W

    pltpu.sync_copy(idx_hbm.at[wid], idx_v)

    for b in range(_NBUF):
        pltpu.async_copy(codebook_hbm.at[idx_v.at[pl.ds(b * _ROWS, _ROWS)]],
                         rows_v.at[b], gsem[b])

    def _chunk(j, b, fb):
        pltpu.make_async_copy(
            codebook_hbm.at[pl.ds(0, _ROWS)], rows_v.at[b], gsem[b]).wait()

        @pl.when(j >= _WBUF)
        def _():
            pltpu.make_async_copy(
                feat_v.at[fb], feat_hbm.at[pl.ds(0, _CH_T)], wsem[fb]).wait()

        def _tok(t2, carry):
            for u in range(2):
                t = t2 * 2 + u
                r0 = t * _Q
                acc = [rows_v[b, r0, pl.ds(s * _L, _L)]
                       for s in range(_D // _L)]
                for q in range(1, _Q):
                    for s in range(_D // _L):
                        acc[s] = acc[s] + rows_v[b, r0 + q, pl.ds(s * _L, _L)]
                for s in range(_D // _L):
                    feat_v[fb, t, pl.ds(s * _L, _L)] = acc[s]
            return carry

        lax.fori_loop(0, _CH_T // 2, _tok, 0)

        @pl.when(j + _NBUF < _NCHUNK)
        def _():
            pltpu.async_copy(
                codebook_hbm.at[idx_v.at[pl.ds((j + _NBUF) * _ROWS, _ROWS)]],
                rows_v.at[b], gsem[b])

        pltpu.async_copy(
            feat_v.at[fb], feat_hbm.at[pl.ds(tok0 + j * _CH_T, _CH_T)],
            wsem[fb])

    def _outer(i, carry):
        for b in range(_NBUF):
            _chunk(i * _NBUF + b, b, b % _WBUF)
        return carry

    lax.fori_loop(0, _NCHUNK // _NBUF, _outer, 0)

    for fb in range(_WBUF):
        pltpu.make_async_copy(
            feat_v.at[fb], feat_hbm.at[pl.ds(0, _CH_T)], wsem[fb]).wait()


def _mlp_body(feat_ref, w1_ref, b1_ref, w2_ref, b2_ref, out_ref):
    h = lax.dot_general(feat_ref[...], w1_ref[...], (((1,), (0,)), ((), ())),
                        preferred_element_type=jnp.float32)
    h = jnp.maximum(h + b1_ref[...], 0.0)
    out_ref[...] = lax.dot_general(w2_ref[...], h, (((0,), (1,)), ((), ())),
                                   preferred_element_type=jnp.float32) + b2_ref[...]


_BT = 16384

_mlp = pl.pallas_call(
    _mlp_body,
    grid=(_N // _BT,),
    in_specs=[
        pl.BlockSpec((_BT, _D), lambda i: (i, 0)),
        pl.BlockSpec((_D, _D), lambda i: (0, 0)),
        pl.BlockSpec((1, _D), lambda i: (0, 0)),
        pl.BlockSpec((_D, _MOTION_DIM), lambda i: (0, 0)),
        pl.BlockSpec((_MOTION_DIM, 1), lambda i: (0, 0)),
    ],
    out_specs=pl.BlockSpec((_MOTION_DIM, _BT), lambda i: (0, i)),
    out_shape=jax.ShapeDtypeStruct((_MOTION_DIM, _N), jnp.float32),
)


def kernel(tokens, codebook, W1, b1, W2, b2):
    tok2d = tokens.astype(jnp.int32).reshape(_B, _T * _Q)
    offs = lax.rem(lax.broadcasted_iota(jnp.int32, (_B, _T * _Q), 1), _Q) * _K
    feat = _gather_sum(tok2d + offs, codebook)
    motion_t = _mlp(feat, W1, b1.reshape(1, _D), W2,
                    b2.reshape(_MOTION_DIM, 1))
    return motion_t.T.reshape(_B, _T, _J, 3)
